# baseline (speedup 1.0000x reference)
import functools

import jax
import jax.numpy as jnp
from jax import lax
from jax.experimental import pallas as pl
from jax.experimental.pallas import tpu as pltpu
from jax.experimental.pallas import tpu_sc as plsc

_NC = 2
_NS = 16
_L = 16
_C = 128


@functools.lru_cache(maxsize=None)
def _build(n, d, s, sp):
    w_total = _NC * _NS
    assert s % 8 == 0 and n % 8 == 0
    q = (s // w_total) // 8 * 8
    r = (s - q * w_total) // 8
    swmax = q + (8 if r else 0)
    garb = swmax
    accr = ((swmax + 2 + _L - 1) // _L) * _L
    ow = ((swmax + 1 + 7) // 8) * 8
    assert ow <= sp
    steps = []
    st = 1
    while st < ow:
        st *= 2
    while st >= 1:
        steps.append(st)
        st //= 2

    mesh = plsc.VectorSubcoreMesh(core_axis_name="c", subcore_axis_name="s")

    @functools.partial(
        pl.kernel,
        out_type=jax.ShapeDtypeStruct((s, d), jnp.float32),
        mesh=mesh,
        scratch_types=[
            pltpu.VMEM((ow,), jnp.int32),
            pltpu.VMEM((_C, d), jnp.float32),
            pltpu.VMEM((_C, d), jnp.float32),
            pltpu.VMEM((_C,), jnp.int32),
            pltpu.VMEM((_C,), jnp.int32),
            pltpu.VMEM((_L, d), jnp.float32),
            pltpu.VMEM_SHARED((_NS * accr, d), jnp.float32),
            pltpu.SemaphoreType.DMA,
            pltpu.SemaphoreType.DMA,
            pltpu.SemaphoreType.DMA,
            pltpu.SemaphoreType.DMA,
        ],
        compiler_params=pltpu.CompilerParams(needs_layout_passes=False),
    )
    def seg_kernel(data_hbm, offs_hbm, out_hbm, offs_l, buf0, buf1, idxb0,
                   idxb1, zbuf, acc, gsem0, gsem1, ssem0, ssem1):
        cid = lax.axis_index("c")
        sid = lax.axis_index("s")
        w = cid * _NS + sid
        s0 = w * q + jnp.minimum(w, r) * 8
        nseg = jnp.where(w < r, q + 8, q)
        abase = sid * accr

        for rr in range(_L):
            for cc in range(d // _L):
                zbuf[rr, pl.ds(cc * _L, _L)] = jnp.zeros((_L,), jnp.float32)
        for t in range(accr // _L):
            pltpu.sync_copy(zbuf, acc.at[pl.ds(abase + t * _L, _L)])

        base_a = jnp.minimum(s0, sp - ow)
        pltpu.sync_copy(offs_hbm.at[pl.ds(base_a, ow)], offs_l)
        def _scalar_at(i):
            return plsc.load_gather(
                offs_l, [jnp.full((_L,), i, jnp.int32)])[0]

        rs = _scalar_at(s0 - base_a)
        re = _scalar_at(s0 + nseg - base_a)

        rs8 = (rs // 8) * 8
        nch = (re - rs8 + (_C - 1)) // _C

        def _gather(c, buf, sem):
            base = rs8 + c * _C
            cb = jnp.minimum(base, n - _C)
            return pltpu.make_async_copy(data_hbm.at[pl.ds(cb, _C)], buf, sem)

        def _compute_idx(c, idxb):
            base = rs8 + c * _C
            cb = jnp.minimum(base, n - _C)
            for gi in range(_C // _L):
                g = cb + gi * _L + lax.iota(jnp.int32, _L)
                valid = (g >= jnp.maximum(base, rs)) & (g < re)
                pos = jnp.zeros((_L,), jnp.int32)
                for stp in steps:
                    cand = pos + stp
                    cc2 = jnp.minimum(cand, ow - 1)
                    v = plsc.load_gather(offs_l, [cc2])
                    pos = jnp.where((cand <= ow - 1) & (v <= g), cand, pos)
                aidx = base_a + pos - s0
                idxb[pl.ds(gi * _L, _L)] = jnp.where(valid, aidx, garb) + abase

        @pl.when(nch > 0)
        def _():
            _gather(0, buf0, gsem0).start()

        def chunk_pair(c2, carry):
            for b, bufA, gsemA, idxA, ssemA, bufB, gsemB, idxB, ssemB in (
                (0, buf0, gsem0, idxb0, ssem0, buf1, gsem1, idxb1, ssem1),
                (1, buf1, gsem1, idxb1, ssem1, buf0, gsem0, idxb0, ssem0),
            ):
                cc = 2 * c2 + b

                @pl.when(cc < nch)
                def _(cc=cc, bufA=bufA, gsemA=gsemA, idxA=idxA, ssemA=ssemA,
                      bufB=bufB, gsemB=gsemB, idxB=idxB, ssemB=ssemB):
                    _gather(cc, bufA, gsemA).wait()

                    @pl.when(cc + 1 < nch)
                    def _():
                        @pl.when(cc >= 1)
                        def _():
                            pltpu.make_async_copy(
                                bufB, acc.at[idxB], ssemB).wait()

                        _gather(cc + 1, bufB, gsemB).start()

                    _compute_idx(cc, idxA)
                    pltpu.async_copy(bufA, acc.at[idxA], ssemA, add=True)
            return carry

        lax.fori_loop(0, (nch + 1) // 2, chunk_pair, 0)

        @pl.when(nch >= 2)
        def _():
            pltpu.make_async_copy(buf0, acc.at[idxb0], ssem0).wait()
            pltpu.make_async_copy(buf1, acc.at[idxb1], ssem1).wait()

        @pl.when(nch == 1)
        def _():
            pltpu.make_async_copy(buf0, acc.at[idxb0], ssem0).wait()

        if r:
            @pl.when(w < r)
            def _():
                pltpu.sync_copy(acc.at[pl.ds(abase, q + 8)],
                                out_hbm.at[pl.ds(s0, q + 8)])

        if q:
            @pl.when(w >= r)
            def _():
                pltpu.sync_copy(acc.at[pl.ds(abase, q)],
                                out_hbm.at[pl.ds(s0, q)])

    return seg_kernel


def kernel(data, offsets):
    n, d = data.shape
    s = offsets.shape[0] - 1
    offs = offsets.astype(jnp.int32)
    pad = (-offsets.shape[0]) % 8
    if pad:
        offs = jnp.concatenate([offs, jnp.full((pad,), n, jnp.int32)])
    return _build(n, d, s, int(offs.shape[0]))(data, offs)


# speedup vs baseline: 1.0066x; 1.0066x over previous
import functools

import jax
import jax.numpy as jnp
from jax import lax
from jax.experimental import pallas as pl
from jax.experimental.pallas import tpu as pltpu
from jax.experimental.pallas import tpu_sc as plsc

_NC = 2
_NS = 16
_L = 16
_C = 128
_NB = 4


@functools.lru_cache(maxsize=None)
def _build(n, d, s, sp):
    w_total = _NC * _NS
    assert s % 8 == 0 and n % 8 == 0
    q = (s // w_total) // 8 * 8
    r = (s - q * w_total) // 8
    swmax = q + (8 if r else 0)
    garb = swmax
    accr = ((swmax + 2 + _L - 1) // _L) * _L
    ow = ((swmax + 1 + 7) // 8) * 8
    assert ow <= sp
    steps = []
    st = 1
    while st < ow:
        st *= 2
    while st >= 1:
        steps.append(st)
        st //= 2

    mesh = plsc.VectorSubcoreMesh(core_axis_name="c", subcore_axis_name="s")

    @functools.partial(
        pl.kernel,
        out_type=jax.ShapeDtypeStruct((s, d), jnp.float32),
        mesh=mesh,
        scratch_types=[
            pltpu.VMEM((ow,), jnp.int32),
            [pltpu.VMEM((_C, d), jnp.float32) for _ in range(_NB)],
            pltpu.VMEM((_C,), jnp.int32),
            pltpu.VMEM((_L, d), jnp.float32),
            pltpu.VMEM_SHARED((_NS * accr, d), jnp.float32),
            [pltpu.SemaphoreType.DMA for _ in range(_NB)],
        ],
        compiler_params=pltpu.CompilerParams(needs_layout_passes=False),
    )
    def seg_kernel(data_hbm, offs_hbm, out_hbm, offs_l, bufs, idxb,
                   zbuf, acc, gsems):
        cid = lax.axis_index("c")
        sid = lax.axis_index("s")
        w = cid * _NS + sid
        s0 = w * q + jnp.minimum(w, r) * 8
        nseg = jnp.where(w < r, q + 8, q)
        abase = sid * accr

        for rr in range(_L):
            for cc in range(d // _L):
                zbuf[rr, pl.ds(cc * _L, _L)] = jnp.zeros((_L,), jnp.float32)
        for t in range(accr // _L):
            pltpu.sync_copy(zbuf, acc.at[pl.ds(abase + t * _L, _L)])

        base_a = jnp.minimum(s0, sp - ow)
        pltpu.sync_copy(offs_hbm.at[pl.ds(base_a, ow)], offs_l)
        def _scalar_at(i):
            return plsc.load_gather(
                offs_l, [jnp.full((_L,), i, jnp.int32)])[0]

        rs = _scalar_at(s0 - base_a)
        re = _scalar_at(s0 + nseg - base_a)

        rs8 = (rs // 8) * 8
        nch = (re - rs8 + (_C - 1)) // _C

        def _gather(c, buf, sem):
            base = rs8 + c * _C
            cb = jnp.minimum(base, n - _C)
            return pltpu.make_async_copy(data_hbm.at[pl.ds(cb, _C)], buf, sem)

        def _process(c, buf):
            base = rs8 + c * _C
            cb = jnp.minimum(base, n - _C)
            for gi in range(_C // _L):
                g = cb + gi * _L + lax.iota(jnp.int32, _L)
                valid = (g >= jnp.maximum(base, rs)) & (g < re)
                pos = jnp.zeros((_L,), jnp.int32)
                for stp in steps:
                    cand = pos + stp
                    cc2 = jnp.minimum(cand, ow - 1)
                    v = plsc.load_gather(offs_l, [cc2])
                    pos = jnp.where((cand <= ow - 1) & (v <= g), cand, pos)
                aidx = base_a + pos - s0
                idxb[pl.ds(gi * _L, _L)] = jnp.where(valid, aidx, garb) + abase
            pltpu.sync_copy(buf, acc.at[idxb], add=True)

        for b in range(_NB - 1):
            @pl.when(b < nch)
            def _(b=b):
                _gather(b, bufs[b], gsems[b]).start()

        def chunk_group(cg, carry):
            for b in range(_NB):
                cc = _NB * cg + b

                @pl.when(cc < nch)
                def _(cc=cc, b=b):
                    _gather(cc, bufs[b], gsems[b]).wait()
                    nb = (b + _NB - 1) % _NB

                    @pl.when(cc + _NB - 1 < nch)
                    def _():
                        _gather(cc + _NB - 1, bufs[nb], gsems[nb]).start()

                    _process(cc, bufs[b])
            return carry

        lax.fori_loop(0, (nch + _NB - 1) // _NB, chunk_group, 0)

        if r:
            @pl.when(w < r)
            def _():
                pltpu.sync_copy(acc.at[pl.ds(abase, q + 8)],
                                out_hbm.at[pl.ds(s0, q + 8)])

        if q:
            @pl.when(w >= r)
            def _():
                pltpu.sync_copy(acc.at[pl.ds(abase, q)],
                                out_hbm.at[pl.ds(s0, q)])

    return seg_kernel


def kernel(data, offsets):
    n, d = data.shape
    s = offsets.shape[0] - 1
    offs = offsets.astype(jnp.int32)
    pad = (-offsets.shape[0]) % 8
    if pad:
        offs = jnp.concatenate([offs, jnp.full((pad,), n, jnp.int32)])
    return _build(n, d, s, int(offs.shape[0]))(data, offs)
